# baseline (speedup 1.0000x reference)
import functools

import jax
import jax.numpy as jnp
from jax import lax
from jax.experimental import pallas as pl
from jax.experimental.pallas import tpu as pltpu
from jax.experimental.pallas import tpu_sc as plsc

N_ROWS = 100000
D_FEAT = 128
TOPK = 3125
L = 4096
C = 25
C_PAD = 32
B_GATHER = 3328


def _score_kernel(val_ref, w_ref, b_ref, out_ref):
    pid = pl.program_id(0)
    x = val_ref[...]
    s = jax.nn.sigmoid(
        jnp.dot(x, w_ref[...], preferred_element_type=jnp.float32)
        + b_ref[...]
    )
    row = pid * L + lax.broadcasted_iota(jnp.int32, (L, 1), 0)
    out_ref[...] = jnp.where(row < N_ROWS, s, -1.0).reshape(1, L, 1)


def _scores(val, w, b):
    return pl.pallas_call(
        _score_kernel,
        grid=(C,),
        in_specs=[
            pl.BlockSpec((L, D_FEAT), lambda i: (i, 0)),
            pl.BlockSpec((D_FEAT, 1), lambda i: (0, 0)),
            pl.BlockSpec((1, 1), lambda i: (0, 0)),
        ],
        out_specs=pl.BlockSpec((1, L, 1), lambda i: (i, 0, 0)),
        out_shape=jax.ShapeDtypeStruct((C, L, 1), jnp.float32),
    )(val, w, b.reshape(1, 1))


def _less(ka, va, kb, vb):
    return (ka > kb) | ((ka == kb) & (va < vb))


def _cmpex(keys, vals, j, kk, iota_col, col_asc):
    pk_up = jnp.roll(keys, -j, axis=0)
    pk_dn = jnp.roll(keys, j, axis=0)
    pv_up = jnp.roll(vals, -j, axis=0)
    pv_dn = jnp.roll(vals, j, axis=0)
    is_lo = (iota_col & j) == 0
    pk = jnp.where(is_lo, pk_up, pk_dn)
    pv = jnp.where(is_lo, pv_up, pv_dn)
    if kk:
        dir_up = (iota_col & kk) == 0
        take_min = is_lo == dir_up
    else:
        take_min = is_lo
    take_min = take_min == col_asc
    a_first = _less(keys, vals, pk, pv)
    sel = a_first == take_min
    return jnp.where(sel, keys, pk), jnp.where(sel, vals, pv)


def _topk_kernel(scores_ref, out_s_ref, out_i_ref):
    s = scores_ref[...]
    keys = lax.bitcast_convert_type(s, jnp.int32)
    vals = (
        lax.broadcasted_iota(jnp.int32, (L, C), 1) * L
        + lax.broadcasted_iota(jnp.int32, (L, C), 0)
    )
    keys = jnp.concatenate(
        [keys, jnp.full((L, C_PAD - C), -1, jnp.int32)], axis=1
    )
    vals = jnp.concatenate(
        [vals, jnp.full((L, C_PAD - C), N_ROWS, jnp.int32)], axis=1
    )
    iota_col = lax.broadcasted_iota(jnp.int32, (L, 1), 0)
    col_iota = lax.broadcasted_iota(jnp.int32, (1, C_PAD), 1)

    col_asc = (col_iota & 1) == 0
    kk = 2
    while kk <= L:
        j = kk // 2
        while j >= 1:
            keys, vals = _cmpex(keys, vals, j, kk, iota_col, col_asc)
            j //= 2
        kk *= 2

    step = 1
    while step < C_PAD:
        pk = jnp.roll(keys, -step, axis=1)
        pv = jnp.roll(vals, -step, axis=1)
        a_first = _less(keys, vals, pk, pv)
        keep_a = a_first == col_asc
        keys = jnp.where(keep_a, keys, pk)
        vals = jnp.where(keep_a, vals, pv)
        col_asc = (col_iota & (2 * step)) == 0
        j = L // 2
        while j >= 1:
            keys, vals = _cmpex(keys, vals, j, 0, iota_col, col_asc)
            j //= 2
        step *= 2

    out_s_ref[...] = lax.bitcast_convert_type(keys[:, 0:1], jnp.float32)
    out_i_ref[...] = vals[:, 0:1]


def _topk(scores_lc):
    return pl.pallas_call(
        _topk_kernel,
        out_shape=(
            jax.ShapeDtypeStruct((L, 1), jnp.float32),
            jax.ShapeDtypeStruct((L, 1), jnp.int32),
        ),
    )(scores_lc)


def _sc_gather(table, idx):
    info = plsc.get_sparse_core_info()
    nc, ns = info.num_cores, info.num_subcores
    b_per_w = B_GATHER // (nc * ns)
    mesh = plsc.VectorSubcoreMesh(core_axis_name="c", subcore_axis_name="s")

    @functools.partial(
        pl.kernel,
        mesh=mesh,
        out_type=jax.ShapeDtypeStruct((B_GATHER, D_FEAT), jnp.float32),
        scratch_types=[
            pltpu.VMEM((b_per_w,), jnp.int32),
            pltpu.VMEM((b_per_w, D_FEAT), jnp.float32),
            pltpu.SemaphoreType.DMA,
        ],
    )
    def gather_k(table_hbm, idx_hbm, out_hbm, idx_v, rows_v, sem):
        wid = lax.axis_index("s") * nc + lax.axis_index("c")
        base = wid * b_per_w
        pltpu.sync_copy(idx_hbm.at[pl.ds(base, b_per_w)], idx_v)
        pltpu.async_copy(table_hbm.at[idx_v], rows_v, sem).wait()
        pltpu.sync_copy(rows_v, out_hbm.at[pl.ds(base, b_per_w)])

    return gather_k(table, idx)


def _scale_kernel(rows_ref, s_ref, out_ref):
    out_ref[...] = rows_ref[...] * s_ref[...]


def _scale(rows, s):
    return pl.pallas_call(
        _scale_kernel,
        out_shape=jax.ShapeDtypeStruct((B_GATHER, D_FEAT), jnp.float32),
    )(rows, s)


def kernel(val, W, b, k):
    scores_lc = _scores(val, W, b).reshape(C, L).T
    s_sorted, idx_sorted = _topk(scores_lc)
    idx_flat = idx_sorted[:, 0]
    gathered = _sc_gather(val, idx_flat[:B_GATHER])
    new_val = _scale(gathered, s_sorted[:B_GATHER])
    idx_out = (idx_flat[:TOPK] + (k - TOPK)).astype(jnp.int32)
    return new_val[:TOPK], idx_out


# speedup vs baseline: 2.1121x; 2.1121x over previous
import functools

import jax
import jax.numpy as jnp
from jax import lax
from jax.experimental import pallas as pl
from jax.experimental.pallas import tpu as pltpu
from jax.experimental.pallas import tpu_sc as plsc

N_ROWS = 100000
D_FEAT = 128
TOPK = 3125
L = 4096
C = 25
C_PAD = 32
B_GATHER = 3328


def _score_kernel(val_ref, w_ref, b_ref, out_ref):
    pid = pl.program_id(0)
    x = val_ref[...]
    s = jax.nn.sigmoid(
        jnp.dot(x, w_ref[...], preferred_element_type=jnp.float32)
        + b_ref[...]
    )
    row = pid * L + lax.broadcasted_iota(jnp.int32, (L, 1), 0)
    out_ref[...] = jnp.where(row < N_ROWS, s, -1.0).reshape(1, L, 1)


def _scores(val, w, b):
    return pl.pallas_call(
        _score_kernel,
        grid=(C,),
        in_specs=[
            pl.BlockSpec((L, D_FEAT), lambda i: (i, 0)),
            pl.BlockSpec((D_FEAT, 1), lambda i: (0, 0)),
            pl.BlockSpec((1, 1), lambda i: (0, 0)),
        ],
        out_specs=pl.BlockSpec((1, L, 1), lambda i: (i, 0, 0)),
        out_shape=jax.ShapeDtypeStruct((C, L, 1), jnp.float32),
    )(val, w, b.reshape(1, 1))


def _less(ka, va, kb, vb):
    return (ka > kb) | ((ka == kb) & (va < vb))


def _cmpex(keys, vals, j, kk, iota_col, col_asc):
    pk_up = jnp.roll(keys, -j, axis=1)
    pk_dn = jnp.roll(keys, j, axis=1)
    pv_up = jnp.roll(vals, -j, axis=1)
    pv_dn = jnp.roll(vals, j, axis=1)
    is_lo = (iota_col & j) == 0
    pk = jnp.where(is_lo, pk_up, pk_dn)
    pv = jnp.where(is_lo, pv_up, pv_dn)
    if kk:
        dir_up = (iota_col & kk) == 0
        take_min = is_lo == dir_up
    else:
        take_min = is_lo
    take_min = take_min == col_asc
    a_first = _less(keys, vals, pk, pv)
    sel = a_first == take_min
    return jnp.where(sel, keys, pk), jnp.where(sel, vals, pv)


def _topk_kernel(scores_ref, out_s_ref, out_i_ref):
    s = scores_ref[...]
    keys = lax.bitcast_convert_type(s, jnp.int32)
    vals = (
        lax.broadcasted_iota(jnp.int32, (C, L), 0) * L
        + lax.broadcasted_iota(jnp.int32, (C, L), 1)
    )
    keys = jnp.concatenate(
        [keys, jnp.full((C_PAD - C, L), -1, jnp.int32)], axis=0
    )
    vals = jnp.concatenate(
        [vals, jnp.full((C_PAD - C, L), N_ROWS, jnp.int32)], axis=0
    )
    iota_col = lax.broadcasted_iota(jnp.int32, (1, L), 1)
    col_iota = lax.broadcasted_iota(jnp.int32, (C_PAD, 1), 0)

    col_asc = (col_iota & 1) == 0
    kk = 2
    while kk <= L:
        j = kk // 2
        while j >= 1:
            keys, vals = _cmpex(keys, vals, j, kk, iota_col, col_asc)
            j //= 2
        kk *= 2

    step = 1
    while step < C_PAD:
        pk = jnp.roll(keys, -step, axis=0)
        pv = jnp.roll(vals, -step, axis=0)
        a_first = _less(keys, vals, pk, pv)
        keep_a = a_first == col_asc
        keys = jnp.where(keep_a, keys, pk)
        vals = jnp.where(keep_a, vals, pv)
        col_asc = (col_iota & (2 * step)) == 0
        j = L // 2
        while j >= 1:
            keys, vals = _cmpex(keys, vals, j, 0, iota_col, col_asc)
            j //= 2
        step *= 2

    out_s_ref[...] = lax.bitcast_convert_type(keys[0:1, :], jnp.float32)
    out_i_ref[...] = vals[0:1, :]


def _topk(scores_cl):
    return pl.pallas_call(
        _topk_kernel,
        out_shape=(
            jax.ShapeDtypeStruct((1, L), jnp.float32),
            jax.ShapeDtypeStruct((1, L), jnp.int32),
        ),
    )(scores_cl)


def _sc_gather(table, idx):
    info = plsc.get_sparse_core_info()
    nc, ns = info.num_cores, info.num_subcores
    b_per_w = B_GATHER // (nc * ns)
    mesh = plsc.VectorSubcoreMesh(core_axis_name="c", subcore_axis_name="s")

    @functools.partial(
        pl.kernel,
        mesh=mesh,
        out_type=jax.ShapeDtypeStruct((B_GATHER, D_FEAT), jnp.float32),
        scratch_types=[
            pltpu.VMEM((b_per_w,), jnp.int32),
            pltpu.VMEM((b_per_w, D_FEAT), jnp.float32),
            pltpu.SemaphoreType.DMA,
        ],
    )
    def gather_k(table_hbm, idx_hbm, out_hbm, idx_v, rows_v, sem):
        wid = lax.axis_index("s") * nc + lax.axis_index("c")
        base = wid * b_per_w
        pltpu.sync_copy(idx_hbm.at[pl.ds(base, b_per_w)], idx_v)
        pltpu.async_copy(table_hbm.at[idx_v], rows_v, sem).wait()
        pltpu.sync_copy(rows_v, out_hbm.at[pl.ds(base, b_per_w)])

    return gather_k(table, idx)


def _scale_kernel(rows_ref, s_ref, out_ref):
    out_ref[...] = rows_ref[...] * s_ref[...]


def _scale(rows, s):
    return pl.pallas_call(
        _scale_kernel,
        out_shape=jax.ShapeDtypeStruct((B_GATHER, D_FEAT), jnp.float32),
    )(rows, s)


def kernel(val, W, b, k):
    scores_cl = _scores(val, W, b).reshape(C, L)
    s_sorted, idx_sorted = _topk(scores_cl)
    idx_flat = idx_sorted[0]
    gathered = _sc_gather(val, idx_flat[:B_GATHER])
    new_val = _scale(gathered, s_sorted[0, :B_GATHER, None])
    idx_out = (idx_flat[:TOPK] + (k - TOPK)).astype(jnp.int32)
    return new_val[:TOPK], idx_out
